# Initial kernel scaffold; baseline (speedup 1.0000x reference)
import functools

import jax
import jax.numpy as jnp
from jax import lax
from jax.experimental import pallas as pl
from jax.experimental.pallas import tpu as pltpu
from jax.experimental.pallas import tpu_sc as plsc

N = 10000
E = 320000
D_IN = 128
D_HID = 32

NC, NS = 2, 16
NW = NC * NS
N_PAD = 10240
ROWS_PT = N_PAD // NS
E_PAD = 327680
SUB = 128
SUBS_PT = E_PAD // (NW * SUB)
CHUNK = 8
GATHER_ROWS = CHUNK * SUB

_mesh = plsc.VectorSubcoreMesh(core_axis_name="c", subcore_axis_name="s")


@functools.partial(
    pl.kernel,
    out_type=jax.ShapeDtypeStruct((NC, N_PAD), jnp.float32),
    mesh=_mesh,
    scratch_types=[
        pltpu.VMEM((CHUNK, SUB), jnp.int32),
        pltpu.VMEM((SUB,), jnp.float32),
        pltpu.VMEM((ROWS_PT,), jnp.float32),
        pltpu.VMEM_SHARED((N_PAD,), jnp.float32),
    ],
)
def _deg_sc(dst2d, ones_hbm, zeros_hbm, degp, didx, ones_v, stage, acc):
    c = lax.axis_index("c")
    s = lax.axis_index("s")
    w = c * NS + s
    pltpu.sync_copy(ones_hbm, ones_v)
    pltpu.sync_copy(zeros_hbm, stage)
    pltpu.sync_copy(stage, acc.at[pl.ds(s * ROWS_PT, ROWS_PT)])
    plsc.subcore_barrier()

    def body(i, carry):
        base = w * SUBS_PT + i * CHUNK
        pltpu.sync_copy(dst2d.at[pl.ds(base, CHUNK)], didx)
        for j in range(CHUNK):
            pltpu.sync_copy(ones_v, acc.at[didx.at[j]], add=True)
        return carry

    lax.fori_loop(0, SUBS_PT // CHUNK, body, 0)
    plsc.subcore_barrier()
    pltpu.sync_copy(acc.at[pl.ds(s * ROWS_PT, ROWS_PT)], stage)
    pltpu.sync_copy(stage, degp.at[c, pl.ds(s * ROWS_PT, ROWS_PT)])


@functools.partial(
    pl.kernel,
    out_type=jax.ShapeDtypeStruct((NC, N_PAD, D_HID), jnp.float32),
    mesh=_mesh,
    scratch_types=[
        pltpu.VMEM((CHUNK, SUB), jnp.int32),
        pltpu.VMEM((CHUNK, SUB), jnp.int32),
        pltpu.VMEM((GATHER_ROWS, D_HID), jnp.float32),
        pltpu.VMEM_SHARED((N_PAD, D_HID), jnp.float32),
        pltpu.SemaphoreType.DMA,
    ],
)
def _agg_sc(table, src2d, dst2d, zrows_hbm, aggp, sidx, didx, rows, acc, sem):
    c = lax.axis_index("c")
    s = lax.axis_index("s")
    w = c * NS + s
    pltpu.sync_copy(zrows_hbm, rows.at[pl.ds(0, ROWS_PT)])
    pltpu.sync_copy(rows.at[pl.ds(0, ROWS_PT)], acc.at[pl.ds(s * ROWS_PT, ROWS_PT)])
    plsc.subcore_barrier()

    def body(i, carry):
        base = w * SUBS_PT + i * CHUNK
        pltpu.sync_copy(src2d.at[pl.ds(base, CHUNK)], sidx)
        pltpu.sync_copy(dst2d.at[pl.ds(base, CHUNK)], didx)
        for j in range(CHUNK):
            pltpu.async_copy(
                table.at[sidx.at[j]], rows.at[pl.ds(j * SUB, SUB)], sem
            ).wait()
        for j in range(CHUNK):
            pltpu.sync_copy(
                rows.at[pl.ds(j * SUB, SUB)], acc.at[didx.at[j]], add=True
            )
        return carry

    lax.fori_loop(0, SUBS_PT // CHUNK, body, 0)
    plsc.subcore_barrier()
    pltpu.sync_copy(acc.at[pl.ds(s * ROWS_PT, ROWS_PT)], rows.at[pl.ds(0, ROWS_PT)])
    pltpu.sync_copy(rows.at[pl.ds(0, ROWS_PT)], aggp.at[c, pl.ds(s * ROWS_PT, ROWS_PT)])


def _m1_body(x_ref, w1_ref, degp_ref, h1p_ref, dinv_ref):
    deg = degp_ref[0] + degp_ref[1] + 1.0
    dinv = lax.rsqrt(deg)
    dinv_ref[...] = dinv
    h = jnp.dot(x_ref[...], w1_ref[...], preferred_element_type=jnp.float32)
    h1p_ref[0:N] = h * dinv[0:N]
    h1p_ref[N:N_PAD] = jnp.zeros((N_PAD - N, D_HID), jnp.float32)


_m1 = pl.pallas_call(
    _m1_body,
    out_shape=(
        jax.ShapeDtypeStruct((N_PAD, D_HID), jnp.float32),
        jax.ShapeDtypeStruct((N_PAD, 1), jnp.float32),
    ),
)


def _m2_body(aggp_ref, h1p_ref, dinv_ref, b1_ref, w2_ref, h2p_ref):
    dinv = dinv_ref[0:N]
    agg = aggp_ref[0, 0:N] + aggp_ref[1, 0:N] + h1p_ref[0:N]
    z = jnp.maximum(agg * dinv + b1_ref[...], 0.0)
    h = jnp.dot(z, w2_ref[...], preferred_element_type=jnp.float32)
    h2p_ref[0:N] = h * dinv
    h2p_ref[N:N_PAD] = jnp.zeros((N_PAD - N, D_HID), jnp.float32)


_m2 = pl.pallas_call(
    _m2_body,
    out_shape=jax.ShapeDtypeStruct((N_PAD, D_HID), jnp.float32),
)


def _m3_body(aggp_ref, h2p_ref, dinv_ref, b2_ref, w3_ref, b3_ref, out_ref):
    dinv = dinv_ref[0:N]
    agg = aggp_ref[0, 0:N] + aggp_ref[1, 0:N] + h2p_ref[0:N]
    z = jnp.maximum(agg * dinv + b2_ref[...], 0.0)
    out_ref[...] = jnp.sum(z * w3_ref[...], axis=1, keepdims=True) + b3_ref[...]


_m3 = pl.pallas_call(
    _m3_body,
    out_shape=jax.ShapeDtypeStruct((N, 1), jnp.float32),
)


def kernel(x, edge_index, W1, b1, W2, b2, W3, b3):
    ei = edge_index.astype(jnp.int32)
    pad = jnp.full((E_PAD - E,), N, dtype=jnp.int32)
    src2d = jnp.concatenate([ei[0], pad]).reshape(E_PAD // SUB, SUB)
    dst2d = jnp.concatenate([ei[1], pad]).reshape(E_PAD // SUB, SUB)
    ones = jnp.ones((SUB,), jnp.float32)
    zdeg = jnp.zeros((ROWS_PT,), jnp.float32)
    zrows = jnp.zeros((ROWS_PT, D_HID), jnp.float32)

    degp = _deg_sc(dst2d, ones, zdeg)
    h1p, dinv = _m1(x, W1, degp.reshape(NC, N_PAD, 1))
    agg1 = _agg_sc(h1p, src2d, dst2d, zrows)
    h2p = _m2(agg1, h1p, dinv, b1.reshape(1, D_HID), W2)
    agg2 = _agg_sc(h2p, src2d, dst2d, zrows)
    out = _m3(agg2, h2p, dinv, b2.reshape(1, D_HID), W3.reshape(1, D_HID),
              b3.reshape(1, 1))
    return out


# speedup vs baseline: 22.4085x; 22.4085x over previous
import functools

import jax
import jax.numpy as jnp
from jax import lax
from jax.experimental import pallas as pl
from jax.experimental.pallas import tpu as pltpu
from jax.experimental.pallas import tpu_sc as plsc

N = 10000
E = 320000
D_IN = 128
D_HID = 32

NC, NS = 2, 16
NW = NC * NS
N_PAD = 10240
ROWS_PT = N_PAD // NS
E_PAD = 327680
SUB = 128
SUBS_PT = E_PAD // (NW * SUB)
CHUNK = 8
GATHER_ROWS = CHUNK * SUB

_mesh = plsc.VectorSubcoreMesh(core_axis_name="c", subcore_axis_name="s")


@functools.partial(
    pl.kernel,
    out_type=jax.ShapeDtypeStruct((NC, N_PAD), jnp.float32),
    mesh=_mesh,
    scratch_types=[
        pltpu.VMEM((CHUNK, SUB), jnp.int32),
        pltpu.VMEM((SUB,), jnp.float32),
        pltpu.VMEM((ROWS_PT,), jnp.float32),
        pltpu.VMEM_SHARED((N_PAD,), jnp.float32),
    ],
)
def _deg_sc(dst2d, ones_hbm, zeros_hbm, degp, didx, ones_v, stage, acc):
    c = lax.axis_index("c")
    s = lax.axis_index("s")
    w = c * NS + s
    pltpu.sync_copy(ones_hbm, ones_v)
    pltpu.sync_copy(zeros_hbm, stage)
    pltpu.sync_copy(stage, acc.at[pl.ds(s * ROWS_PT, ROWS_PT)])
    plsc.subcore_barrier()

    def body(i, carry):
        base = w * SUBS_PT + i * CHUNK
        pltpu.sync_copy(dst2d.at[pl.ds(base, CHUNK)], didx)
        for j in range(CHUNK):
            pltpu.sync_copy(ones_v, acc.at[didx.at[j]], add=True)
        return carry

    lax.fori_loop(0, SUBS_PT // CHUNK, body, 0)
    plsc.subcore_barrier()
    pltpu.sync_copy(acc.at[pl.ds(s * ROWS_PT, ROWS_PT)], stage)
    pltpu.sync_copy(stage, degp.at[c, pl.ds(s * ROWS_PT, ROWS_PT)])


@functools.partial(
    pl.kernel,
    out_type=jax.ShapeDtypeStruct((NC, N_PAD, D_HID), jnp.float32),
    mesh=_mesh,
    scratch_types=[
        pltpu.VMEM((CHUNK, SUB), jnp.int32),
        pltpu.VMEM((CHUNK, SUB), jnp.int32),
        pltpu.VMEM((GATHER_ROWS, D_HID), jnp.float32),
        pltpu.VMEM_SHARED((N_PAD, D_HID), jnp.float32),
        pltpu.SemaphoreType.DMA,
    ],
    compiler_params=pltpu.CompilerParams(use_tc_tiling_on_sc=False),
)
def _agg_sc(table, src2d, dst2d, zrows_hbm, aggp, sidx, didx, rows, acc, sem):
    c = lax.axis_index("c")
    s = lax.axis_index("s")
    w = c * NS + s
    pltpu.sync_copy(zrows_hbm, rows.at[pl.ds(0, ROWS_PT)])
    pltpu.sync_copy(rows.at[pl.ds(0, ROWS_PT)], acc.at[pl.ds(s * ROWS_PT, ROWS_PT)])
    plsc.subcore_barrier()

    def body(i, carry):
        base = w * SUBS_PT + i * CHUNK
        pltpu.sync_copy(src2d.at[pl.ds(base, CHUNK)], sidx)
        pltpu.sync_copy(dst2d.at[pl.ds(base, CHUNK)], didx)
        for j in range(CHUNK):
            pltpu.async_copy(
                table.at[sidx.at[j]], rows.at[pl.ds(j * SUB, SUB)], sem
            ).wait()
        for j in range(CHUNK):
            pltpu.sync_copy(
                rows.at[pl.ds(j * SUB, SUB)], acc.at[didx.at[j]], add=True
            )
        return carry

    lax.fori_loop(0, SUBS_PT // CHUNK, body, 0)
    plsc.subcore_barrier()
    pltpu.sync_copy(acc.at[pl.ds(s * ROWS_PT, ROWS_PT)], rows.at[pl.ds(0, ROWS_PT)])
    pltpu.sync_copy(rows.at[pl.ds(0, ROWS_PT)], aggp.at[c, pl.ds(s * ROWS_PT, ROWS_PT)])


def _m1_body(x_ref, w1_ref, degp_ref, h1p_ref, dinv_ref):
    deg = degp_ref[0] + degp_ref[1] + 1.0
    dinv = lax.rsqrt(deg)
    dinv_ref[...] = dinv
    h = jnp.dot(x_ref[...], w1_ref[...], preferred_element_type=jnp.float32)
    h1p_ref[0:N] = h * dinv[0:N]
    h1p_ref[N:N_PAD] = jnp.zeros((N_PAD - N, D_HID), jnp.float32)


_m1 = pl.pallas_call(
    _m1_body,
    out_shape=(
        jax.ShapeDtypeStruct((N_PAD, D_HID), jnp.float32),
        jax.ShapeDtypeStruct((N_PAD, 1), jnp.float32),
    ),
)


def _m2_body(aggp_ref, h1p_ref, dinv_ref, b1_ref, w2_ref, h2p_ref):
    dinv = dinv_ref[0:N]
    agg = aggp_ref[0, 0:N] + aggp_ref[1, 0:N] + h1p_ref[0:N]
    z = jnp.maximum(agg * dinv + b1_ref[...], 0.0)
    h = jnp.dot(z, w2_ref[...], preferred_element_type=jnp.float32)
    h2p_ref[0:N] = h * dinv
    h2p_ref[N:N_PAD] = jnp.zeros((N_PAD - N, D_HID), jnp.float32)


_m2 = pl.pallas_call(
    _m2_body,
    out_shape=jax.ShapeDtypeStruct((N_PAD, D_HID), jnp.float32),
)


def _m3_body(aggp_ref, h2p_ref, dinv_ref, b2_ref, w3_ref, b3_ref, out_ref):
    dinv = dinv_ref[0:N]
    agg = aggp_ref[0, 0:N] + aggp_ref[1, 0:N] + h2p_ref[0:N]
    z = jnp.maximum(agg * dinv + b2_ref[...], 0.0)
    out_ref[...] = jnp.sum(z * w3_ref[...], axis=1, keepdims=True) + b3_ref[...]


_m3 = pl.pallas_call(
    _m3_body,
    out_shape=jax.ShapeDtypeStruct((N, 1), jnp.float32),
)


def kernel(x, edge_index, W1, b1, W2, b2, W3, b3):
    ei = edge_index.astype(jnp.int32)
    pad = jnp.full((E_PAD - E,), N, dtype=jnp.int32)
    src2d = jnp.concatenate([ei[0], pad]).reshape(E_PAD // SUB, SUB)
    dst2d = jnp.concatenate([ei[1], pad]).reshape(E_PAD // SUB, SUB)
    ones = jnp.ones((SUB,), jnp.float32)
    zdeg = jnp.zeros((ROWS_PT,), jnp.float32)
    zrows = jnp.zeros((ROWS_PT, D_HID), jnp.float32)

    degp = _deg_sc(dst2d, ones, zdeg)
    h1p, dinv = _m1(x, W1, degp.reshape(NC, N_PAD, 1))
    agg1 = _agg_sc(h1p, src2d, dst2d, zrows)
    h2p = _m2(agg1, h1p, dinv, b1.reshape(1, D_HID), W2)
    agg2 = _agg_sc(h2p, src2d, dst2d, zrows)
    out = _m3(agg2, h2p, dinv, b2.reshape(1, D_HID), W3.reshape(1, D_HID),
              b3.reshape(1, 1))
    return out


# speedup vs baseline: 29.2169x; 1.3038x over previous
import functools

import jax
import jax.numpy as jnp
from jax import lax
from jax.experimental import pallas as pl
from jax.experimental.pallas import tpu as pltpu
from jax.experimental.pallas import tpu_sc as plsc

N = 10000
E = 320000
D_IN = 128
D_HID = 32

NC, NS = 2, 16
NW = NC * NS
N_PAD = 10240
ROWS_PT = N_PAD // NS
E_PAD = 327680
SUB = 128
SUBS_PT = E_PAD // (NW * SUB)
CHUNK = 8
GATHER_ROWS = CHUNK * SUB

_mesh = plsc.VectorSubcoreMesh(core_axis_name="c", subcore_axis_name="s")


@functools.partial(
    pl.kernel,
    out_type=jax.ShapeDtypeStruct((NC, N_PAD), jnp.float32),
    mesh=_mesh,
    scratch_types=[
        pltpu.VMEM((SUBS_PT, SUB), jnp.int32),
        pltpu.VMEM((SUB,), jnp.float32),
        pltpu.VMEM((ROWS_PT,), jnp.float32),
        pltpu.VMEM_SHARED((N_PAD,), jnp.float32),
        pltpu.SemaphoreType.DMA,
    ],
)
def _deg_sc(dst2d, ones_hbm, zeros_hbm, degp, didx, ones_v, stage, acc, sem):
    c = lax.axis_index("c")
    s = lax.axis_index("s")
    w = c * NS + s
    pltpu.sync_copy(dst2d.at[pl.ds(w * SUBS_PT, SUBS_PT)], didx)
    pltpu.sync_copy(ones_hbm, ones_v)
    pltpu.sync_copy(zeros_hbm, stage)
    pltpu.sync_copy(stage, acc.at[pl.ds(s * ROWS_PT, ROWS_PT)])
    plsc.subcore_barrier()

    copies = [
        pltpu.async_copy(ones_v, acc.at[didx.at[i]], sem, add=True)
        for i in range(SUBS_PT)
    ]
    for cp in copies:
        cp.wait()
    plsc.subcore_barrier()
    pltpu.sync_copy(acc.at[pl.ds(s * ROWS_PT, ROWS_PT)], stage)
    pltpu.sync_copy(stage, degp.at[c, pl.ds(s * ROWS_PT, ROWS_PT)])


@functools.partial(
    pl.kernel,
    out_type=jax.ShapeDtypeStruct((NC, N_PAD, D_HID), jnp.float32),
    mesh=_mesh,
    scratch_types=[
        pltpu.VMEM((SUBS_PT, SUB), jnp.int32),
        pltpu.VMEM((SUBS_PT, SUB), jnp.int32),
        pltpu.VMEM((2, GATHER_ROWS, D_HID), jnp.float32),
        pltpu.VMEM_SHARED((N_PAD, D_HID), jnp.float32),
        pltpu.SemaphoreType.DMA,
        pltpu.SemaphoreType.DMA,
        pltpu.SemaphoreType.DMA,
        pltpu.SemaphoreType.DMA,
    ],
    compiler_params=pltpu.CompilerParams(use_tc_tiling_on_sc=False),
)
def _agg_sc(table, src2d, dst2d, zrows_hbm, aggp, sidx, didx, rows, acc,
            gsem0, gsem1, ssem0, ssem1):
    c = lax.axis_index("c")
    s = lax.axis_index("s")
    w = c * NS + s
    gsem = (gsem0, gsem1)
    ssem = (ssem0, ssem1)
    nch = SUBS_PT // CHUNK
    pltpu.sync_copy(src2d.at[pl.ds(w * SUBS_PT, SUBS_PT)], sidx)
    pltpu.sync_copy(dst2d.at[pl.ds(w * SUBS_PT, SUBS_PT)], didx)
    pltpu.sync_copy(zrows_hbm, rows.at[0, pl.ds(0, ROWS_PT)])
    pltpu.sync_copy(rows.at[0, pl.ds(0, ROWS_PT)],
                    acc.at[pl.ds(s * ROWS_PT, ROWS_PT)])
    plsc.subcore_barrier()

    def fire_gathers(i, b):
        return [
            pltpu.async_copy(
                table.at[sidx.at[i * CHUNK + j]],
                rows.at[b, pl.ds(j * SUB, SUB)],
                gsem[b],
            )
            for j in range(CHUNK)
        ]

    def fire_scatters(i, b):
        return [
            pltpu.async_copy(
                rows.at[b, pl.ds(j * SUB, SUB)],
                acc.at[didx.at[i * CHUNK + j]],
                ssem[b],
                add=True,
            )
            for j in range(CHUNK)
        ]

    gcp = {0: fire_gathers(0, 0)}
    scp = {}
    for i in range(nch):
        b = i % 2
        if i + 1 < nch:
            if i >= 1:
                for cp in scp.pop(i - 1):
                    cp.wait()
            gcp[i + 1] = fire_gathers(i + 1, 1 - b)
        for cp in gcp.pop(i):
            cp.wait()
        scp[i] = fire_scatters(i, b)
    for i in list(scp):
        for cp in scp.pop(i):
            cp.wait()
    plsc.subcore_barrier()
    pltpu.sync_copy(acc.at[pl.ds(s * ROWS_PT, ROWS_PT)],
                    rows.at[0, pl.ds(0, ROWS_PT)])
    pltpu.sync_copy(rows.at[0, pl.ds(0, ROWS_PT)],
                    aggp.at[c, pl.ds(s * ROWS_PT, ROWS_PT)])


def _m1_body(x_ref, w1_ref, degp_ref, h1p_ref, dinv_ref):
    deg = degp_ref[0] + degp_ref[1] + 1.0
    dinv = lax.rsqrt(deg)
    dinv_ref[...] = dinv
    h = jnp.dot(x_ref[...], w1_ref[...], preferred_element_type=jnp.float32)
    h1p_ref[0:N] = h * dinv[0:N]
    h1p_ref[N:N_PAD] = jnp.zeros((N_PAD - N, D_HID), jnp.float32)


_m1 = pl.pallas_call(
    _m1_body,
    out_shape=(
        jax.ShapeDtypeStruct((N_PAD, D_HID), jnp.float32),
        jax.ShapeDtypeStruct((N_PAD, 1), jnp.float32),
    ),
)


def _m2_body(aggp_ref, h1p_ref, dinv_ref, b1_ref, w2_ref, h2p_ref):
    dinv = dinv_ref[0:N]
    agg = aggp_ref[0, 0:N] + aggp_ref[1, 0:N] + h1p_ref[0:N]
    z = jnp.maximum(agg * dinv + b1_ref[...], 0.0)
    h = jnp.dot(z, w2_ref[...], preferred_element_type=jnp.float32)
    h2p_ref[0:N] = h * dinv
    h2p_ref[N:N_PAD] = jnp.zeros((N_PAD - N, D_HID), jnp.float32)


_m2 = pl.pallas_call(
    _m2_body,
    out_shape=jax.ShapeDtypeStruct((N_PAD, D_HID), jnp.float32),
)


def _m3_body(aggp_ref, h2p_ref, dinv_ref, b2_ref, w3_ref, b3_ref, out_ref):
    dinv = dinv_ref[0:N]
    agg = aggp_ref[0, 0:N] + aggp_ref[1, 0:N] + h2p_ref[0:N]
    z = jnp.maximum(agg * dinv + b2_ref[...], 0.0)
    out_ref[...] = jnp.sum(z * w3_ref[...], axis=1, keepdims=True) + b3_ref[...]


_m3 = pl.pallas_call(
    _m3_body,
    out_shape=jax.ShapeDtypeStruct((N, 1), jnp.float32),
)


def kernel(x, edge_index, W1, b1, W2, b2, W3, b3):
    ei = edge_index.astype(jnp.int32)
    pad = jnp.full((E_PAD - E,), N, dtype=jnp.int32)
    src2d = jnp.concatenate([ei[0], pad]).reshape(E_PAD // SUB, SUB)
    dst2d = jnp.concatenate([ei[1], pad]).reshape(E_PAD // SUB, SUB)
    ones = jnp.ones((SUB,), jnp.float32)
    zdeg = jnp.zeros((ROWS_PT,), jnp.float32)
    zrows = jnp.zeros((ROWS_PT, D_HID), jnp.float32)

    degp = _deg_sc(dst2d, ones, zdeg)
    h1p, dinv = _m1(x, W1, degp.reshape(NC, N_PAD, 1))
    agg1 = _agg_sc(h1p, src2d, dst2d, zrows)
    h2p = _m2(agg1, h1p, dinv, b1.reshape(1, D_HID), W2)
    agg2 = _agg_sc(h2p, src2d, dst2d, zrows)
    out = _m3(agg2, h2p, dinv, b2.reshape(1, D_HID), W3.reshape(1, D_HID),
              b3.reshape(1, 1))
    return out


# speedup vs baseline: 44.2828x; 1.5157x over previous
import functools

import jax
import jax.numpy as jnp
from jax import lax
from jax.experimental import pallas as pl
from jax.experimental.pallas import tpu as pltpu
from jax.experimental.pallas import tpu_sc as plsc

N = 10000
E = 320000
D_IN = 128
D_HID = 32

NC, NS = 2, 16
NW = NC * NS
N_PAD = 10240
ROWS_PT = N_PAD // NS
E_PAD = 327680
SUB = 128
SUBS_PT = E_PAD // (NW * SUB)
CHUNK = 8
GATHER_ROWS = CHUNK * SUB

_mesh = plsc.VectorSubcoreMesh(core_axis_name="c", subcore_axis_name="s")


@functools.partial(
    pl.kernel,
    out_type=jax.ShapeDtypeStruct((NC, N_PAD), jnp.float32),
    mesh=_mesh,
    scratch_types=[
        pltpu.VMEM((SUBS_PT, SUB), jnp.int32),
        pltpu.VMEM((SUB,), jnp.float32),
        pltpu.VMEM((ROWS_PT,), jnp.float32),
        pltpu.VMEM_SHARED((N_PAD,), jnp.float32),
        pltpu.SemaphoreType.DMA,
    ],
)
def _deg_sc(dst2d, ones_hbm, zeros_hbm, degp, didx, ones_v, stage, acc, sem):
    c = lax.axis_index("c")
    s = lax.axis_index("s")
    w = c * NS + s
    pltpu.sync_copy(dst2d.at[pl.ds(w * SUBS_PT, SUBS_PT)], didx)
    pltpu.sync_copy(ones_hbm, ones_v)
    pltpu.sync_copy(zeros_hbm, stage)
    pltpu.sync_copy(stage, acc.at[pl.ds(s * ROWS_PT, ROWS_PT)])
    plsc.subcore_barrier()

    copies = [
        pltpu.async_copy(ones_v, acc.at[didx.at[i]], sem, add=True)
        for i in range(SUBS_PT)
    ]
    for cp in copies:
        cp.wait()
    plsc.subcore_barrier()
    pltpu.sync_copy(acc.at[pl.ds(s * ROWS_PT, ROWS_PT)], stage)
    pltpu.sync_copy(stage, degp.at[c, pl.ds(s * ROWS_PT, ROWS_PT)])


@functools.partial(
    pl.kernel,
    out_type=jax.ShapeDtypeStruct((NC, N_PAD, D_HID), jnp.float32),
    mesh=_mesh,
    scratch_types=[
        pltpu.VMEM((SUBS_PT, SUB), jnp.int32),
        pltpu.VMEM((SUBS_PT, SUB), jnp.int32),
        pltpu.VMEM((2, GATHER_ROWS, D_HID), jnp.float32),
        pltpu.VMEM_SHARED((N_PAD, D_HID), jnp.float32),
        pltpu.VMEM_SHARED((N_PAD, D_HID), jnp.float32),
        pltpu.SemaphoreType.DMA,
        pltpu.SemaphoreType.DMA,
        pltpu.SemaphoreType.DMA,
        pltpu.SemaphoreType.DMA,
    ],
    compiler_params=pltpu.CompilerParams(use_tc_tiling_on_sc=False),
)
def _agg_sc(table, src2d, dst2d, zrows_hbm, aggp, sidx, didx, rows, acc,
            tbl, gsem0, gsem1, ssem0, ssem1):
    c = lax.axis_index("c")
    s = lax.axis_index("s")
    w = c * NS + s
    gsem = (gsem0, gsem1)
    ssem = (ssem0, ssem1)
    nch = SUBS_PT // CHUNK
    pltpu.sync_copy(src2d.at[pl.ds(w * SUBS_PT, SUBS_PT)], sidx)
    pltpu.sync_copy(dst2d.at[pl.ds(w * SUBS_PT, SUBS_PT)], didx)
    pltpu.sync_copy(table.at[pl.ds(s * ROWS_PT, ROWS_PT)],
                    rows.at[1, pl.ds(0, ROWS_PT)])
    pltpu.sync_copy(rows.at[1, pl.ds(0, ROWS_PT)],
                    tbl.at[pl.ds(s * ROWS_PT, ROWS_PT)])
    pltpu.sync_copy(zrows_hbm, rows.at[0, pl.ds(0, ROWS_PT)])
    pltpu.sync_copy(rows.at[0, pl.ds(0, ROWS_PT)],
                    acc.at[pl.ds(s * ROWS_PT, ROWS_PT)])
    plsc.subcore_barrier()

    def fire_gathers(i, b):
        return [
            pltpu.async_copy(
                tbl.at[sidx.at[i * CHUNK + j]],
                rows.at[b, pl.ds(j * SUB, SUB)],
                gsem[b],
            )
            for j in range(CHUNK)
        ]

    def fire_scatters(i, b):
        return [
            pltpu.async_copy(
                rows.at[b, pl.ds(j * SUB, SUB)],
                acc.at[didx.at[i * CHUNK + j]],
                ssem[b],
                add=True,
            )
            for j in range(CHUNK)
        ]

    gcp = {0: fire_gathers(0, 0)}
    scp = {}
    for i in range(nch):
        b = i % 2
        if i + 1 < nch:
            if i >= 1:
                for cp in scp.pop(i - 1):
                    cp.wait()
            gcp[i + 1] = fire_gathers(i + 1, 1 - b)
        for cp in gcp.pop(i):
            cp.wait()
        scp[i] = fire_scatters(i, b)
    for i in list(scp):
        for cp in scp.pop(i):
            cp.wait()
    plsc.subcore_barrier()
    pltpu.sync_copy(acc.at[pl.ds(s * ROWS_PT, ROWS_PT)],
                    rows.at[0, pl.ds(0, ROWS_PT)])
    pltpu.sync_copy(rows.at[0, pl.ds(0, ROWS_PT)],
                    aggp.at[c, pl.ds(s * ROWS_PT, ROWS_PT)])


def _m1_body(x_ref, w1_ref, degp_ref, h1p_ref, dinv_ref):
    deg = degp_ref[0] + degp_ref[1] + 1.0
    dinv = lax.rsqrt(deg)
    dinv_ref[...] = dinv
    h = jnp.dot(x_ref[...], w1_ref[...], preferred_element_type=jnp.float32)
    h1p_ref[0:N] = h * dinv[0:N]
    h1p_ref[N:N_PAD] = jnp.zeros((N_PAD - N, D_HID), jnp.float32)


_m1 = pl.pallas_call(
    _m1_body,
    out_shape=(
        jax.ShapeDtypeStruct((N_PAD, D_HID), jnp.float32),
        jax.ShapeDtypeStruct((N_PAD, 1), jnp.float32),
    ),
)


def _m2_body(aggp_ref, h1p_ref, dinv_ref, b1_ref, w2_ref, h2p_ref):
    dinv = dinv_ref[0:N]
    agg = aggp_ref[0, 0:N] + aggp_ref[1, 0:N] + h1p_ref[0:N]
    z = jnp.maximum(agg * dinv + b1_ref[...], 0.0)
    h = jnp.dot(z, w2_ref[...], preferred_element_type=jnp.float32)
    h2p_ref[0:N] = h * dinv
    h2p_ref[N:N_PAD] = jnp.zeros((N_PAD - N, D_HID), jnp.float32)


_m2 = pl.pallas_call(
    _m2_body,
    out_shape=jax.ShapeDtypeStruct((N_PAD, D_HID), jnp.float32),
)


def _m3_body(aggp_ref, h2p_ref, dinv_ref, b2_ref, w3_ref, b3_ref, out_ref):
    dinv = dinv_ref[0:N]
    agg = aggp_ref[0, 0:N] + aggp_ref[1, 0:N] + h2p_ref[0:N]
    z = jnp.maximum(agg * dinv + b2_ref[...], 0.0)
    out_ref[...] = jnp.sum(z * w3_ref[...], axis=1, keepdims=True) + b3_ref[...]


_m3 = pl.pallas_call(
    _m3_body,
    out_shape=jax.ShapeDtypeStruct((N, 1), jnp.float32),
)


def kernel(x, edge_index, W1, b1, W2, b2, W3, b3):
    ei = edge_index.astype(jnp.int32)
    pad = jnp.full((E_PAD - E,), N, dtype=jnp.int32)
    src2d = jnp.concatenate([ei[0], pad]).reshape(E_PAD // SUB, SUB)
    dst2d = jnp.concatenate([ei[1], pad]).reshape(E_PAD // SUB, SUB)
    ones = jnp.ones((SUB,), jnp.float32)
    zdeg = jnp.zeros((ROWS_PT,), jnp.float32)
    zrows = jnp.zeros((ROWS_PT, D_HID), jnp.float32)

    degp = _deg_sc(dst2d, ones, zdeg)
    h1p, dinv = _m1(x, W1, degp.reshape(NC, N_PAD, 1))
    agg1 = _agg_sc(h1p, src2d, dst2d, zrows)
    h2p = _m2(agg1, h1p, dinv, b1.reshape(1, D_HID), W2)
    agg2 = _agg_sc(h2p, src2d, dst2d, zrows)
    out = _m3(agg2, h2p, dinv, b2.reshape(1, D_HID), W3.reshape(1, D_HID),
              b3.reshape(1, 1))
    return out


# speedup vs baseline: 55.2675x; 1.2481x over previous
import functools

import jax
import jax.numpy as jnp
from jax import lax
from jax.experimental import pallas as pl
from jax.experimental.pallas import tpu as pltpu
from jax.experimental.pallas import tpu_sc as plsc

N = 10000
E = 320000
D_IN = 128
D_HID = 32

NC, NS = 2, 16
NW = NC * NS
N_PAD = 10240
PK = N_PAD // 4
ROWS_PT = N_PAD // NS
E_PAD = 327680
SUB = 128
SUBS_PT = E_PAD // (NW * SUB)
CHUNK = 5
GATHER_ROWS = CHUNK * SUB
PAD_IDX = (N % PK) * 4 + N // PK

_mesh = plsc.VectorSubcoreMesh(core_axis_name="c", subcore_axis_name="s")


@functools.partial(
    pl.kernel,
    out_type=jax.ShapeDtypeStruct((NC, N_PAD), jnp.float32),
    mesh=_mesh,
    scratch_types=[
        pltpu.VMEM((SUBS_PT, SUB), jnp.int32),
        pltpu.VMEM((SUB,), jnp.float32),
        pltpu.VMEM((ROWS_PT,), jnp.float32),
        pltpu.VMEM_SHARED((N_PAD,), jnp.float32),
        pltpu.SemaphoreType.DMA,
    ],
    compiler_params=pltpu.CompilerParams(use_tc_tiling_on_sc=False),
)
def _deg_sc(dst2d, ones_hbm, zeros_hbm, degp, didx, ones_v, stage, acc, sem):
    c = lax.axis_index("c")
    s = lax.axis_index("s")
    w = c * NS + s
    pltpu.sync_copy(dst2d.at[pl.ds(w * SUBS_PT, SUBS_PT)], didx)
    pltpu.sync_copy(ones_hbm, ones_v)
    pltpu.sync_copy(zeros_hbm, stage)
    pltpu.sync_copy(stage, acc.at[pl.ds(s * ROWS_PT, ROWS_PT)])
    plsc.subcore_barrier()

    copies = [
        pltpu.async_copy(ones_v, acc.at[didx.at[i]], sem, add=True)
        for i in range(SUBS_PT)
    ]
    for cp in copies:
        cp.wait()
    plsc.subcore_barrier()
    pltpu.sync_copy(acc.at[pl.ds(s * ROWS_PT, ROWS_PT)], stage)
    pltpu.sync_copy(stage, degp.at[c, pl.ds(s * ROWS_PT, ROWS_PT)])


_AGG_KW = dict(
    out_type=(
        jax.ShapeDtypeStruct((NC, N_PAD, D_HID), jnp.float32),
        jax.ShapeDtypeStruct((N_PAD, D_HID), jnp.float32),
    ),
    mesh=_mesh,
    scratch_types=[
        pltpu.VMEM((SUBS_PT, SUB), jnp.int32),
        pltpu.VMEM((SUBS_PT, SUB), jnp.int32),
        pltpu.VMEM((GATHER_ROWS, D_HID), jnp.float32),
        pltpu.VMEM((GATHER_ROWS, D_HID), jnp.float32),
        pltpu.VMEM((ROWS_PT, D_HID), jnp.float32),
        pltpu.VMEM((ROWS_PT,), jnp.float32),
        pltpu.VMEM_SHARED((N_PAD, D_HID), jnp.float32),
        pltpu.VMEM_SHARED((N_PAD, D_HID), jnp.float32),
        pltpu.SemaphoreType.DMA,
        pltpu.SemaphoreType.DMA,
        pltpu.SemaphoreType.DMA,
        pltpu.SemaphoreType.DMA,
    ],
    compiler_params=pltpu.CompilerParams(use_tc_tiling_on_sc=False),
)


def _agg_body(tbl_hbm, dinv_hbm, src2d, dst2d, qpart, selfterm,
            sidx, didx, rows_a, rows_b, selfb, dinv_v, acc, tbl,
            gsem0, gsem1, ssem0, ssem1):
    rows = (rows_a, rows_b)
    c = lax.axis_index("c")
    s = lax.axis_index("s")
    w = c * NS + s
    gsem = (gsem0, gsem1)
    ssem = (ssem0, ssem1)
    nch = SUBS_PT // CHUNK
    r0 = s * ROWS_PT

    pltpu.sync_copy(src2d.at[pl.ds(w * SUBS_PT, SUBS_PT)], sidx)
    pltpu.sync_copy(dst2d.at[pl.ds(w * SUBS_PT, SUBS_PT)], didx)
    pltpu.sync_copy(tbl_hbm.at[pl.ds(r0, ROWS_PT)], rows_b.at[pl.ds(0, ROWS_PT)])
    pltpu.sync_copy(dinv_hbm.at[pl.ds(r0, ROWS_PT)], dinv_v)

    zero16 = jnp.zeros((16,), jnp.float32)

    def scale_block(q, carry):
        dvec = dinv_v[pl.ds(q * 16, 16)]
        for j in range(16):
            r = q * 16 + j
            d = jnp.broadcast_to(dvec[j], (16,))
            lo = rows_b[r, pl.ds(0, 16)] * d
            hi = rows_b[r, pl.ds(16, 16)] * d
            rows_b[r, pl.ds(0, 16)] = lo
            rows_b[r, pl.ds(16, 16)] = hi
            selfb[r, pl.ds(0, 16)] = lo * d
            selfb[r, pl.ds(16, 16)] = hi * d
            rows_a[r, pl.ds(0, 16)] = zero16
            rows_a[r, pl.ds(16, 16)] = zero16
        return carry

    lax.fori_loop(0, ROWS_PT // 16, scale_block, 0)
    pltpu.sync_copy(rows_b.at[pl.ds(0, ROWS_PT)], tbl.at[pl.ds(r0, ROWS_PT)])
    pltpu.sync_copy(rows_a.at[pl.ds(0, ROWS_PT)], acc.at[pl.ds(r0, ROWS_PT)])

    @pl.when(c == 0)
    def _():
        pltpu.sync_copy(selfb, selfterm.at[pl.ds(r0, ROWS_PT)])

    plsc.subcore_barrier()

    def fire_gathers(i, b):
        return [
            pltpu.async_copy(
                tbl.at[sidx.at[i * CHUNK + j]],
                rows[b].at[pl.ds(j * SUB, SUB)],
                gsem[b],
            )
            for j in range(CHUNK)
        ]

    def fire_scatters(i, b):
        return [
            pltpu.async_copy(
                rows[b].at[pl.ds(j * SUB, SUB)],
                acc.at[didx.at[i * CHUNK + j]],
                ssem[b],
                add=True,
            )
            for j in range(CHUNK)
        ]

    gcp = {0: fire_gathers(0, 0)}
    scp = {}
    for i in range(nch):
        b = i % 2
        if i + 1 < nch:
            if i >= 1:
                for cp in scp.pop(i - 1):
                    cp.wait()
            gcp[i + 1] = fire_gathers(i + 1, 1 - b)
        for cp in gcp.pop(i):
            cp.wait()
        scp[i] = fire_scatters(i, b)
    for i in list(scp):
        for cp in scp.pop(i):
            cp.wait()
    plsc.subcore_barrier()

    pltpu.sync_copy(acc.at[pl.ds(r0, ROWS_PT)], rows_a.at[pl.ds(0, ROWS_PT)])

    def post_block(q, carry):
        dvec = dinv_v[pl.ds(q * 16, 16)]
        for j in range(16):
            r = q * 16 + j
            d = jnp.broadcast_to(dvec[j], (16,))
            rows_a[r, pl.ds(0, 16)] = rows_a[r, pl.ds(0, 16)] * d
            rows_a[r, pl.ds(16, 16)] = rows_a[r, pl.ds(16, 16)] * d
        return carry

    lax.fori_loop(0, ROWS_PT // 16, post_block, 0)
    pltpu.sync_copy(rows_a.at[pl.ds(0, ROWS_PT)],
                    qpart.at[c, pl.ds(r0, ROWS_PT)])


_agg_sc = pl.kernel(_agg_body, **_AGG_KW)


def _m0_body(x_ref, w1_ref, h1pk_ref):
    w1 = w1_ref[...]
    for k in range(3):
        h1pk_ref[:, k * 32:(k + 1) * 32] = jnp.dot(
            x_ref[k * PK:(k + 1) * PK, :], w1,
            preferred_element_type=jnp.float32)
    tail = N - 3 * PK
    h1pk_ref[0:tail, 96:128] = jnp.dot(
        x_ref[3 * PK:N, :], w1, preferred_element_type=jnp.float32)
    h1pk_ref[tail:PK, 96:128] = jnp.zeros((PK - tail, 32), jnp.float32)


_m0 = pl.pallas_call(
    _m0_body,
    out_shape=jax.ShapeDtypeStruct((PK, 128), jnp.float32),
)


def _m1_body(degp_ref, dinv_ref):
    dinv_ref[...] = lax.rsqrt(degp_ref[0] + degp_ref[1] + 1.0)


_m1 = pl.pallas_call(
    _m1_body,
    out_shape=jax.ShapeDtypeStruct((N_PAD // 128, 128), jnp.float32),
)


def _m2_body(q_ref, self_ref, b1pk_ref, w2bd_ref, h2pk_ref):
    z = jnp.maximum(q_ref[0] + q_ref[1] + self_ref[...] + b1pk_ref[...], 0.0)
    h2pk_ref[...] = jnp.dot(z, w2bd_ref[...], preferred_element_type=jnp.float32)


_m2 = pl.pallas_call(
    _m2_body,
    out_shape=jax.ShapeDtypeStruct((PK, 128), jnp.float32),
)


def _m3_body(q_ref, self_ref, b2pk_ref, w3blk_ref, b3_ref, out4_ref):
    z = jnp.maximum(q_ref[0] + q_ref[1] + self_ref[...] + b2pk_ref[...], 0.0)
    out4_ref[...] = jnp.dot(z, w3blk_ref[...],
                            preferred_element_type=jnp.float32) + b3_ref[...]


_m3 = pl.pallas_call(
    _m3_body,
    out_shape=jax.ShapeDtypeStruct((PK, 4), jnp.float32),
)


def kernel(x, edge_index, W1, b1, W2, b2, W3, b3):
    ei = edge_index.astype(jnp.int32)
    pe = (ei % PK) * 4 + ei // PK
    pad = jnp.full((E_PAD - E,), PAD_IDX, dtype=jnp.int32)
    src2d = jnp.concatenate([pe[0], pad]).reshape(E_PAD // SUB, SUB)
    dst2d = jnp.concatenate([pe[1], pad]).reshape(E_PAD // SUB, SUB)
    ones = jnp.ones((SUB,), jnp.float32)
    zdeg = jnp.zeros((ROWS_PT,), jnp.float32)
    b1pk = jnp.tile(b1, 4).reshape(1, 128)
    b2pk = jnp.tile(b2, 4).reshape(1, 128)
    w2bd = jnp.kron(jnp.eye(4, dtype=jnp.float32), W2)
    w3blk = jnp.kron(jnp.eye(4, dtype=jnp.float32), W3)

    degp = _deg_sc(dst2d, ones, zdeg)
    h1pk = _m0(x, W1)
    dinv128 = _m1(degp.reshape(NC, N_PAD // 128, 128))
    dinv = dinv128.reshape(N_PAD)
    q1, self1 = _agg_sc(h1pk.reshape(N_PAD, D_HID), dinv, src2d, dst2d)
    h2pk = _m2(q1.reshape(NC, PK, 128), self1.reshape(PK, 128), b1pk, w2bd)
    q2, self2 = _agg_sc(h2pk.reshape(N_PAD, D_HID), dinv, src2d, dst2d)
    out4 = _m3(q2.reshape(NC, PK, 128), self2.reshape(PK, 128), b2pk, w3blk,
               b3.reshape(1, 1))
    return out4.T.reshape(N_PAD, 1)[:N]
